# baseline (speedup 1.0000x reference)
import functools

import jax
import jax.numpy as jnp
from jax import lax
from jax.experimental import pallas as pl
from jax.experimental.pallas import tpu as pltpu
import jax.experimental.pallas.tpu_sc as plsc

M = 1000000
B = 16384
D = 64
PW = 16

NW = 16
BPW = B // NW
CH = 128
NCH = BPW // CH


def _sc_body(mem, idx2, pos16, zrows, val,
             out, postab,
             idxv, repv, sidx, srep, spos, sgot,
             zv, valv, mrows, arows, orows,
             acc):
    w = lax.axis_index("s")
    rowbase = w * NCH
    base = w * BPW

    pltpu.sync_copy(idx2.at[pl.ds(rowbase, NCH)], idxv)
    pltpu.sync_copy(zrows, zv)

    for j in range(NCH):
        for l in range(CH // 16):
            sl = pl.ds(l * 16, 16)
            sidx[sl] = idxv[j, sl]
        pltpu.sync_copy(pos16.at[pl.ds(base + j * CH, CH)], spos)
        pltpu.sync_copy(spos, postab.at[sidx])
    plsc.subcore_barrier()

    zcol = jnp.zeros((16,), jnp.int32)
    for j in range(NCH):
        for l in range(CH // 16):
            sl = pl.ds(l * 16, 16)
            sidx[sl] = idxv[j, sl]
        pltpu.sync_copy(postab.at[sidx], sgot)
        for l in range(CH // 16):
            rows = lax.iota(jnp.int32, 16) + l * 16
            rep16 = plsc.load_gather(sgot, [rows, zcol])
            repv[j, pl.ds(l * 16, 16)] = rep16
            srep[pl.ds(l * 16, 16)] = rep16
        pltpu.sync_copy(zv, acc.at[srep])
    plsc.subcore_barrier()

    for j in range(NCH):
        for l in range(CH // 16):
            sl = pl.ds(l * 16, 16)
            srep[sl] = repv[j, sl]
        pltpu.sync_copy(val.at[pl.ds(base + j * CH, CH)], valv)
        pltpu.sync_copy(valv, acc.at[srep], add=True)
    plsc.subcore_barrier()

    for j in range(NCH):
        for l in range(CH // 16):
            sl = pl.ds(l * 16, 16)
            sidx[sl] = idxv[j, sl]
            srep[sl] = repv[j, sl]
        pltpu.sync_copy(mem.at[sidx], mrows)
        pltpu.sync_copy(acc.at[srep], arows)

        def add_row(r, carry):
            for c in range(D // 16):
                sl = pl.ds(c * 16, 16)
                orows[r, sl] = mrows[r, sl] + arows[r, sl]
            return carry

        lax.fori_loop(0, CH, add_row, 0)
        pltpu.sync_copy(orows, out.at[pl.ds(base + j * CH, CH)])


def kernel(mem, idx, val):
    idx2 = idx.astype(jnp.int32).reshape(B // CH, CH)
    pos16 = jnp.broadcast_to(
        lax.iota(jnp.int32, B)[:, None], (B, PW)).astype(jnp.int32)
    zrows = jnp.zeros((CH, D), jnp.float32)
    mesh = plsc.VectorSubcoreMesh(
        core_axis_name="c", subcore_axis_name="s", num_cores=1)
    run = pl.kernel(
        _sc_body,
        out_type=(
            jax.ShapeDtypeStruct((B, D), jnp.float32),
            jax.ShapeDtypeStruct((M, PW), jnp.int32),
        ),
        mesh=mesh,
        compiler_params=pltpu.CompilerParams(
            use_tc_tiling_on_sc=False, needs_layout_passes=False),
        scratch_types=[
            pltpu.VMEM((NCH, CH), jnp.int32),
            pltpu.VMEM((NCH, CH), jnp.int32),
            pltpu.VMEM((CH,), jnp.int32),
            pltpu.VMEM((CH,), jnp.int32),
            pltpu.VMEM((CH, PW), jnp.int32),
            pltpu.VMEM((CH, PW), jnp.int32),
            pltpu.VMEM((CH, D), jnp.float32),
            pltpu.VMEM((CH, D), jnp.float32),
            pltpu.VMEM((CH, D), jnp.float32),
            pltpu.VMEM((CH, D), jnp.float32),
            pltpu.VMEM((CH, D), jnp.float32),
            pltpu.VMEM_SHARED((B, D), jnp.float32),
        ],
    )
    out, _ = run(mem, idx2, pos16, zrows, val)
    return out
